# baseline (speedup 1.0000x reference)
import functools

import jax
import jax.numpy as jnp
from jax import lax
from jax.experimental import pallas as pl
from jax.experimental.pallas import tpu as pltpu
from jax.experimental.pallas import tpu_sc as plsc

_NC = 2
_NS = 16
_NW = _NC * _NS
_NBUF = 4


def _build_transpose(V, D, CH):
    def body(x_ref, o_ref):
        o_ref[...] = x_ref[...].T

    return pl.pallas_call(
        body,
        grid=(pl.cdiv(V, CH),),
        in_specs=[pl.BlockSpec((D, CH), lambda i: (0, i))],
        out_specs=pl.BlockSpec((CH, D), lambda i: (i, 0)),
        out_shape=jax.ShapeDtypeStruct((V, D), jnp.float32),
    )


def _build_sc_phase(B, L, D, n_split):
    EPW = B // _NW
    HALF = D // 2
    pair = L % 8 != 0
    if pair:
        NG = EPW // 2
        ROWS = 2 * L
    else:
        NG = EPW
        ROWS = L
    if n_split == 2:
        R0 = (ROWS // 2) // 8 * 8
        R1 = ROWS - R0
    else:
        R0, R1 = ROWS, 0

    mesh = plsc.VectorSubcoreMesh(core_axis_name="c", subcore_axis_name="s")

    @functools.partial(
        pl.kernel,
        mesh=mesh,
        compiler_params=pltpu.CompilerParams(use_tc_tiling_on_sc=False),
        out_type=jax.ShapeDtypeStruct((B * D,), jnp.float32),
        scratch_types=[
            pltpu.VMEM((EPW * L,), jnp.int32),
            pltpu.VMEM((_NBUF, ROWS, D), jnp.float32),
            pltpu.VMEM((EPW * D,), jnp.float32),
            pltpu.SemaphoreType.DMA((_NBUF,)),
        ],
    )
    def sc_phase(ix_hbm, tab_hbm, out_hbm, idx, bufs, acc, sem):
        c = lax.axis_index("c")
        s = lax.axis_index("s")
        wid = s * _NC + c
        base_e = wid * EPW

        pltpu.sync_copy(ix_hbm.at[pl.ds(base_e * L, EPW * L)], idx)

        zeros = jnp.zeros((HALF,), jnp.float32)

        def dma(g, b):
            off = pl.multiple_of(g * ROWS, 8)
            cps = [pltpu.make_async_copy(
                tab_hbm.at[idx.at[pl.ds(off, R0)]],
                bufs.at[b, pl.ds(0, R0)], sem.at[b])]
            if R1:
                cps.append(pltpu.make_async_copy(
                    tab_hbm.at[idx.at[pl.ds(off + R0, R1)]],
                    bufs.at[b, pl.ds(R0, R1)], sem.at[b]))
            return cps

        for b in range(_NBUF):
            for cp in dma(b, b):
                cp.start()

        def group(i, _):
            for b in range(_NBUF):
                g = i * _NBUF + b
                for cp in dma(g, b):
                    cp.wait()
                n_el = 2 if pair else 1
                for sub in range(n_el):
                    r0 = sub * L
                    lo = [zeros] * 4
                    hi = [zeros] * 4
                    for l in range(0, L, 4):
                        for u in range(4):
                            lo[u] = lo[u] + bufs[b, r0 + l + u, pl.ds(0, HALF)]
                            hi[u] = hi[u] + bufs[b, r0 + l + u,
                                                 pl.ds(HALF, HALF)]
                    off = pl.multiple_of((n_el * g + sub) * D, 8)
                    acc[pl.ds(off, HALF)] = (lo[0] + lo[1]) + (lo[2] + lo[3])
                    acc[pl.ds(off + HALF, HALF)] = \
                        (hi[0] + hi[1]) + (hi[2] + hi[3])
                g_next = g + _NBUF

                @pl.when(g_next < NG)
                def _():
                    for cp in dma(g_next, b):
                        cp.start()
            return _

        lax.fori_loop(0, NG // _NBUF, group, None)

        pltpu.sync_copy(acc, out_hbm.at[pl.ds(base_e * D, EPW * D)])

    return sc_phase


def _build_combine(B, LQ, LD, D):
    def body(q_ref, d_ref, o_ref):
        q = q_ref[...] * (1.0 / LQ)
        d = d_ref[...] * (1.0 / LD)
        dot = jnp.sum(q * d, axis=1, keepdims=True)
        nq = jnp.sqrt(jnp.sum(q * q, axis=1, keepdims=True))
        nd = jnp.sqrt(jnp.sum(d * d, axis=1, keepdims=True))
        o_ref[...] = dot / (jnp.maximum(nq, 1e-12) * jnp.maximum(nd, 1e-12))

    return pl.pallas_call(
        body,
        out_shape=jax.ShapeDtypeStruct((B, 1), jnp.float32),
    )


@functools.lru_cache(maxsize=None)
def _build(B, LQ, LD, D, VQ, VD):
    return (_build_transpose(VQ, D, 4096), _build_transpose(VD, D, 4096),
            _build_sc_phase(B, LQ, D, 1), _build_sc_phase(B, LD, D, 2),
            _build_combine(B, LQ, LD, D))


def kernel(qs, ds, rels, q_table, d_table):
    B, LQ = qs.shape
    LD = ds.shape[1]
    VQ, D = q_table.shape
    VD = d_table.shape[0]
    t_q, t_d, sc_q, sc_d, combine = _build(B, LQ, LD, D, VQ, VD)
    d_rm = t_d(d_table.T)
    d_sum = sc_d(ds.reshape(-1), d_rm)
    q_rm = t_q(q_table.T)
    q_sum = sc_q(qs.reshape(-1), q_rm)
    sims = combine(q_sum.reshape(B, D), d_sum.reshape(B, D))
    return sims.reshape(B)


# speedup vs baseline: 1.0879x; 1.0879x over previous
import functools

import jax
import jax.numpy as jnp
from jax import lax
from jax.experimental import pallas as pl
from jax.experimental.pallas import tpu as pltpu
from jax.experimental.pallas import tpu_sc as plsc

_NC = 2
_NS = 16
_NW = _NC * _NS
_NBUF = 4


def _build_transpose(V, D, CH):
    def body(x_ref, o_ref):
        i = lax.broadcasted_iota(jnp.int32, (D, D), 0)
        j = lax.broadcasted_iota(jnp.int32, (D, D), 1)
        eye = jnp.where(i == j, 1.0, 0.0).astype(jnp.float32)
        o_ref[...] = lax.dot_general(
            x_ref[...], eye, (((0,), (0,)), ((), ())),
            preferred_element_type=jnp.float32)

    return pl.pallas_call(
        body,
        grid=(pl.cdiv(V, CH),),
        in_specs=[pl.BlockSpec((D, CH), lambda i: (0, i))],
        out_specs=pl.BlockSpec((CH, D), lambda i: (i, 0)),
        out_shape=jax.ShapeDtypeStruct((V, D), jnp.float32),
    )


def _build_sc_phase(B, L, D, n_split):
    EPW = B // _NW
    HALF = D // 2
    pair = L % 8 != 0
    if pair:
        NG = EPW // 2
        ROWS = 2 * L
    else:
        NG = EPW
        ROWS = L
    if n_split == 2:
        R0 = (ROWS // 2) // 8 * 8
        R1 = ROWS - R0
    else:
        R0, R1 = ROWS, 0

    mesh = plsc.VectorSubcoreMesh(core_axis_name="c", subcore_axis_name="s")

    @functools.partial(
        pl.kernel,
        mesh=mesh,
        compiler_params=pltpu.CompilerParams(use_tc_tiling_on_sc=False),
        out_type=jax.ShapeDtypeStruct((B * D,), jnp.float32),
        scratch_types=[
            pltpu.VMEM((EPW * L,), jnp.int32),
            pltpu.VMEM((_NBUF, ROWS, D), jnp.float32),
            pltpu.VMEM((EPW * D,), jnp.float32),
            pltpu.SemaphoreType.DMA((_NBUF,)),
        ],
    )
    def sc_phase(ix_hbm, tab_hbm, out_hbm, idx, bufs, acc, sem):
        c = lax.axis_index("c")
        s = lax.axis_index("s")
        wid = s * _NC + c
        base_e = wid * EPW

        pltpu.sync_copy(ix_hbm.at[pl.ds(base_e * L, EPW * L)], idx)

        zeros = jnp.zeros((HALF,), jnp.float32)

        def dma(g, b):
            off = pl.multiple_of(g * ROWS, 8)
            cps = [pltpu.make_async_copy(
                tab_hbm.at[idx.at[pl.ds(off, R0)]],
                bufs.at[b, pl.ds(0, R0)], sem.at[b])]
            if R1:
                cps.append(pltpu.make_async_copy(
                    tab_hbm.at[idx.at[pl.ds(off + R0, R1)]],
                    bufs.at[b, pl.ds(R0, R1)], sem.at[b]))
            return cps

        for b in range(_NBUF):
            for cp in dma(b, b):
                cp.start()

        def group(i, _):
            for b in range(_NBUF):
                g = i * _NBUF + b
                for cp in dma(g, b):
                    cp.wait()
                n_el = 2 if pair else 1
                for sub in range(n_el):
                    r0 = sub * L
                    lo = [zeros] * 4
                    hi = [zeros] * 4
                    for l in range(0, L, 4):
                        for u in range(4):
                            lo[u] = lo[u] + bufs[b, r0 + l + u, pl.ds(0, HALF)]
                            hi[u] = hi[u] + bufs[b, r0 + l + u,
                                                 pl.ds(HALF, HALF)]
                    off = pl.multiple_of((n_el * g + sub) * D, 8)
                    acc[pl.ds(off, HALF)] = (lo[0] + lo[1]) + (lo[2] + lo[3])
                    acc[pl.ds(off + HALF, HALF)] = \
                        (hi[0] + hi[1]) + (hi[2] + hi[3])
                g_next = g + _NBUF

                @pl.when(g_next < NG)
                def _():
                    for cp in dma(g_next, b):
                        cp.start()
            return _

        lax.fori_loop(0, NG // _NBUF, group, None)

        pltpu.sync_copy(acc, out_hbm.at[pl.ds(base_e * D, EPW * D)])

    return sc_phase


def _build_combine(B, LQ, LD, D):
    def body(q_ref, d_ref, o_ref):
        q = q_ref[...] * (1.0 / LQ)
        d = d_ref[...] * (1.0 / LD)
        dot = jnp.sum(q * d, axis=1, keepdims=True)
        nq = jnp.sqrt(jnp.sum(q * q, axis=1, keepdims=True))
        nd = jnp.sqrt(jnp.sum(d * d, axis=1, keepdims=True))
        o_ref[...] = dot / (jnp.maximum(nq, 1e-12) * jnp.maximum(nd, 1e-12))

    return pl.pallas_call(
        body,
        out_shape=jax.ShapeDtypeStruct((B, 1), jnp.float32),
    )


@functools.lru_cache(maxsize=None)
def _build(B, LQ, LD, D, VQ, VD):
    return (_build_transpose(VQ, D, 8192), _build_transpose(VD, D, 8192),
            _build_sc_phase(B, LQ, D, 1), _build_sc_phase(B, LD, D, 2),
            _build_combine(B, LQ, LD, D))


def kernel(qs, ds, rels, q_table, d_table):
    B, LQ = qs.shape
    LD = ds.shape[1]
    VQ, D = q_table.shape
    VD = d_table.shape[0]
    t_q, t_d, sc_q, sc_d, combine = _build(B, LQ, LD, D, VQ, VD)
    d_rm = t_d(d_table.T)
    d_sum = sc_d(ds.reshape(-1), d_rm)
    q_rm = t_q(q_table.T)
    q_sum = sc_q(qs.reshape(-1), q_rm)
    sims = combine(q_sum.reshape(B, D), d_sum.reshape(B, D))
    return sims.reshape(B)


# speedup vs baseline: 1.9751x; 1.8155x over previous
import functools

import jax
import jax.numpy as jnp
from jax import lax
from jax.experimental import pallas as pl
from jax.experimental.pallas import tpu as pltpu
from jax.experimental.pallas import tpu_sc as plsc

_NC = 2
_NS = 16
_NW = _NC * _NS
_NBUF = 4
_CH = 2048
_BLK = 4 * _CH


def _build_transpose(V, D):
    NBLK = pl.cdiv(V, _BLK)

    def body(x_ref, o_ref):
        parts = [x_ref[:, j * _CH:(j + 1) * _CH].T for j in range(4)]
        o_ref[...] = jnp.concatenate(parts, axis=1)

    return pl.pallas_call(
        body,
        grid=(NBLK,),
        in_specs=[pl.BlockSpec((D, _BLK), lambda i: (0, i))],
        out_specs=pl.BlockSpec((_CH, 4 * D), lambda i: (i, 0)),
        out_shape=jax.ShapeDtypeStruct((NBLK * _CH, 4 * D), jnp.float32),
    )


def _build_sc_phase(B, L, D, n_split, VR):
    EPW = B // _NW
    HALF = D // 2
    pair = L % 8 != 0
    NG = EPW // 2 if pair else EPW
    ROWS = 2 * L if pair else L
    if n_split == 2:
        R0 = (ROWS // 2) // 8 * 8
        R1 = ROWS - R0
    else:
        R0, R1 = ROWS, 0
    NIDX = EPW * L
    SHJ = _CH.bit_length() - 1

    mesh = plsc.VectorSubcoreMesh(core_axis_name="c", subcore_axis_name="s")

    @functools.partial(
        pl.kernel,
        mesh=mesh,
        compiler_params=pltpu.CompilerParams(use_tc_tiling_on_sc=False),
        out_type=jax.ShapeDtypeStruct((B * D,), jnp.float32),
        scratch_types=[
            pltpu.VMEM((NIDX,), jnp.int32),
            pltpu.VMEM((_NBUF, ROWS, D), jnp.float32),
            pltpu.VMEM((EPW * D,), jnp.float32),
            pltpu.SemaphoreType.DMA((_NBUF,)),
        ],
    )
    def sc_phase(ix_hbm, tab_hbm, out_hbm, idx, bufs, acc, sem):
        c = lax.axis_index("c")
        s = lax.axis_index("s")
        wid = s * _NC + c
        base_e = wid * EPW

        pltpu.sync_copy(ix_hbm.at[pl.ds(base_e * L, NIDX)], idx)

        def xform(k, _):
            for u in range(4):
                off = pl.multiple_of((k * 4 + u) * 16, 8)
                v = idx[pl.ds(off, 16)]
                idx[pl.ds(off, 16)] = ((v & jnp.int32(-_BLK))
                                       + ((v & jnp.int32(_CH - 1)) << 2)
                                       + ((v >> SHJ) & 3))
            return _

        lax.fori_loop(0, NIDX // 64, xform, None)

        zeros = jnp.zeros((HALF,), jnp.float32)

        def dma(g, b):
            off = pl.multiple_of(g * ROWS, 8)
            cps = [pltpu.make_async_copy(
                tab_hbm.at[idx.at[pl.ds(off, R0)]],
                bufs.at[b, pl.ds(0, R0)], sem.at[b])]
            if R1:
                cps.append(pltpu.make_async_copy(
                    tab_hbm.at[idx.at[pl.ds(off + R0, R1)]],
                    bufs.at[b, pl.ds(R0, R1)], sem.at[b]))
            return cps

        for b in range(_NBUF):
            for cp in dma(b, b):
                cp.start()

        def group(i, _):
            for b in range(_NBUF):
                g = i * _NBUF + b
                for cp in dma(g, b):
                    cp.wait()
                n_el = 2 if pair else 1
                for sub in range(n_el):
                    r0 = sub * L
                    lo = [zeros] * 4
                    hi = [zeros] * 4
                    for l in range(0, L, 4):
                        for u in range(4):
                            lo[u] = lo[u] + bufs[b, r0 + l + u, pl.ds(0, HALF)]
                            hi[u] = hi[u] + bufs[b, r0 + l + u,
                                                 pl.ds(HALF, HALF)]
                    off = pl.multiple_of((n_el * g + sub) * D, 8)
                    acc[pl.ds(off, HALF)] = (lo[0] + lo[1]) + (lo[2] + lo[3])
                    acc[pl.ds(off + HALF, HALF)] = \
                        (hi[0] + hi[1]) + (hi[2] + hi[3])
                g_next = g + _NBUF

                @pl.when(g_next < NG)
                def _():
                    for cp in dma(g_next, b):
                        cp.start()
            return _

        lax.fori_loop(0, NG // _NBUF, group, None)

        pltpu.sync_copy(acc, out_hbm.at[pl.ds(base_e * D, EPW * D)])

    return sc_phase


def _build_combine(B, LQ, LD, D):
    def body(q_ref, d_ref, o_ref):
        q = q_ref[...] * (1.0 / LQ)
        d = d_ref[...] * (1.0 / LD)
        dot = jnp.sum(q * d, axis=1, keepdims=True)
        nq = jnp.sqrt(jnp.sum(q * q, axis=1, keepdims=True))
        nd = jnp.sqrt(jnp.sum(d * d, axis=1, keepdims=True))
        o_ref[...] = dot / (jnp.maximum(nq, 1e-12) * jnp.maximum(nd, 1e-12))

    return pl.pallas_call(
        body,
        out_shape=jax.ShapeDtypeStruct((B, 1), jnp.float32),
    )


@functools.lru_cache(maxsize=None)
def _build(B, LQ, LD, D, VQ, VD):
    vrq = pl.cdiv(VQ, _BLK) * _BLK
    vrd = pl.cdiv(VD, _BLK) * _BLK
    return (_build_transpose(VQ, D), _build_transpose(VD, D),
            _build_sc_phase(B, LQ, D, 1, vrq),
            _build_sc_phase(B, LD, D, 2, vrd),
            _build_combine(B, LQ, LD, D))


def kernel(qs, ds, rels, q_table, d_table):
    B, LQ = qs.shape
    LD = ds.shape[1]
    VQ, D = q_table.shape
    VD = d_table.shape[0]
    t_q, t_d, sc_q, sc_d, combine = _build(B, LQ, LD, D, VQ, VD)
    d_rm = t_d(d_table.T).reshape(-1, D)
    d_sum = sc_d(ds.reshape(-1), d_rm)
    q_rm = t_q(q_table.T).reshape(-1, D)
    q_sum = sc_q(qs.reshape(-1), q_rm)
    sims = combine(q_sum.reshape(B, D), d_sum.reshape(B, D))
    return sims.reshape(B)


# speedup vs baseline: 2.4724x; 1.2518x over previous
import functools

import jax
import jax.numpy as jnp
from jax import lax
from jax.experimental import pallas as pl
from jax.experimental.pallas import tpu as pltpu
from jax.experimental.pallas import tpu_sc as plsc

_NC = 2
_NS = 16
_NW = _NC * _NS
_NBUF = 4
_CH = 8192
_BLK = 4 * _CH


def _build_transpose(V, D, with_dep=False):
    NBLK = pl.cdiv(V, _BLK)

    def body(x_ref, *rest):
        o_ref = rest[-1]
        parts = [x_ref[:, j * _CH:(j + 1) * _CH].T for j in range(4)]
        o_ref[...] = jnp.concatenate(parts, axis=1)

    in_specs = [pl.BlockSpec((D, _BLK), lambda i: (0, i))]
    if with_dep:
        in_specs.append(pl.BlockSpec((8, D), lambda i: (0, 0)))

    return pl.pallas_call(
        body,
        grid=(NBLK,),
        in_specs=in_specs,
        out_specs=pl.BlockSpec((_CH, 4 * D), lambda i: (i, 0)),
        out_shape=jax.ShapeDtypeStruct((NBLK * _CH, 4 * D), jnp.float32),
    )


def _build_sc_phase(B, L, D, n_split, VR):
    EPW = B // _NW
    HALF = D // 2
    pair = L % 8 != 0
    NG = EPW // 2 if pair else EPW
    ROWS = 2 * L if pair else L
    if n_split == 2:
        R0 = (ROWS // 2) // 8 * 8
        R1 = ROWS - R0
    else:
        R0, R1 = ROWS, 0
    NIDX = EPW * L
    SHJ = _CH.bit_length() - 1

    mesh = plsc.VectorSubcoreMesh(core_axis_name="c", subcore_axis_name="s")

    @functools.partial(
        pl.kernel,
        mesh=mesh,
        compiler_params=pltpu.CompilerParams(use_tc_tiling_on_sc=False),
        out_type=jax.ShapeDtypeStruct((B * D,), jnp.float32),
        scratch_types=[
            pltpu.VMEM((NIDX,), jnp.int32),
            pltpu.VMEM((_NBUF, ROWS, D), jnp.float32),
            pltpu.VMEM((EPW * D,), jnp.float32),
            pltpu.SemaphoreType.DMA((_NBUF,)),
        ],
    )
    def sc_phase(ix_hbm, tab_hbm, out_hbm, idx, bufs, acc, sem):
        c = lax.axis_index("c")
        s = lax.axis_index("s")
        wid = s * _NC + c
        base_e = wid * EPW

        pltpu.sync_copy(ix_hbm.at[pl.ds(base_e * L, NIDX)], idx)

        def xform(k, _):
            for u in range(4):
                off = pl.multiple_of((k * 4 + u) * 16, 8)
                v = idx[pl.ds(off, 16)]
                idx[pl.ds(off, 16)] = ((v & jnp.int32(-_BLK))
                                       + ((v & jnp.int32(_CH - 1)) << 2)
                                       + ((v >> SHJ) & 3))
            return _

        lax.fori_loop(0, NIDX // 64, xform, None)

        zeros = jnp.zeros((HALF,), jnp.float32)

        def dma(g, b):
            off = pl.multiple_of(g * ROWS, 8)
            cps = [pltpu.make_async_copy(
                tab_hbm.at[idx.at[pl.ds(off, R0)]],
                bufs.at[b, pl.ds(0, R0)], sem.at[b])]
            if R1:
                cps.append(pltpu.make_async_copy(
                    tab_hbm.at[idx.at[pl.ds(off + R0, R1)]],
                    bufs.at[b, pl.ds(R0, R1)], sem.at[b]))
            return cps

        for b in range(_NBUF):
            for cp in dma(b, b):
                cp.start()

        def group(i, _):
            for b in range(_NBUF):
                g = i * _NBUF + b
                for cp in dma(g, b):
                    cp.wait()
                n_el = 2 if pair else 1
                for sub in range(n_el):
                    r0 = sub * L
                    lo = [zeros] * 4
                    hi = [zeros] * 4
                    for l in range(0, L, 4):
                        for u in range(4):
                            lo[u] = lo[u] + bufs[b, r0 + l + u, pl.ds(0, HALF)]
                            hi[u] = hi[u] + bufs[b, r0 + l + u,
                                                 pl.ds(HALF, HALF)]
                    off = pl.multiple_of((n_el * g + sub) * D, 8)
                    acc[pl.ds(off, HALF)] = (lo[0] + lo[1]) + (lo[2] + lo[3])
                    acc[pl.ds(off + HALF, HALF)] = \
                        (hi[0] + hi[1]) + (hi[2] + hi[3])
                g_next = g + _NBUF

                @pl.when(g_next < NG)
                def _():
                    for cp in dma(g_next, b):
                        cp.start()
            return _

        lax.fori_loop(0, NG // _NBUF, group, None)

        pltpu.sync_copy(acc, out_hbm.at[pl.ds(base_e * D, EPW * D)])

    return sc_phase


def _build_combine(B, LQ, LD, D):
    def body(d_ref, q_ref, o_ref):
        q = q_ref[...] * (1.0 / LQ)
        d = d_ref[...] * (1.0 / LD)
        dot = jnp.sum(q * d, axis=1, keepdims=True)
        nq = jnp.sqrt(jnp.sum(q * q, axis=1, keepdims=True))
        nd = jnp.sqrt(jnp.sum(d * d, axis=1, keepdims=True))
        o_ref[...] = dot / (jnp.maximum(nq, 1e-12) * jnp.maximum(nd, 1e-12))

    return pl.pallas_call(
        body,
        out_shape=jax.ShapeDtypeStruct((B, 1), jnp.float32),
    )


@functools.lru_cache(maxsize=None)
def _build(B, LQ, LD, D, VQ, VD):
    vrq = pl.cdiv(VQ, _BLK) * _BLK
    vrd = pl.cdiv(VD, _BLK) * _BLK
    return (_build_transpose(VQ, D), _build_transpose(VD, D),
            _build_sc_phase(B, LQ, D, 1, vrq),
            _build_sc_phase(B, LD, D, 2, vrd),
            _build_combine(B, LQ, LD, D))


def kernel(qs, ds, rels, q_table, d_table):
    B, LQ = qs.shape
    LD = ds.shape[1]
    VQ, D = q_table.shape
    VD = d_table.shape[0]
    t_q, t_d, sc_q, sc_d, combine = _build(B, LQ, LD, D, VQ, VD)
    d_rm = t_d(d_table.T).reshape(-1, D)
    d_sum = sc_d(ds.reshape(-1), d_rm)
    q_rm = t_q(q_table.T).reshape(-1, D)
    q_sum = sc_q(qs.reshape(-1), q_rm)
    sims = combine(d_sum.reshape(B, D), q_sum.reshape(B, D))
    return sims.reshape(B)
